# Initial kernel scaffold; baseline (speedup 1.0000x reference)
import jax
import jax.numpy as jnp
from jax.experimental import pallas as pl


def kernel(x_drug, x_disorder, edge_index_drug_to_disorder, edge_index_disorder_to_drug, edge_label_index, params):
    raise NotImplementedError("write your pallas kernel here")


# speedup vs baseline: 1.9389x; 1.9389x over previous
import functools

import jax
import jax.numpy as jnp
from jax import lax
from jax.experimental import pallas as pl
from jax.experimental.pallas import tpu as pltpu
from jax.experimental.pallas import tpu_sc as plsc

F32 = jnp.float32

NC = 2
NS = 16
NW = NC * NS
CW = 32
K_E = 128
R_ACC = 50176
RPT = R_ACC // NS
ZB = 448
BM = 2000


def _mm_bias(x, w, b):
    m, k = x.shape
    n = w.shape[1]

    def body(x_ref, w_ref, b_ref, o_ref):
        o_ref[...] = jnp.dot(x_ref[...], w_ref[...],
                             preferred_element_type=F32) + b_ref[...]

    return pl.pallas_call(
        body,
        grid=(m // BM,),
        in_specs=[
            pl.BlockSpec((BM, k), lambda i: (i, 0)),
            pl.BlockSpec((k, n), lambda i: (0, 0)),
            pl.BlockSpec((1, n), lambda i: (0, 0)),
        ],
        out_specs=pl.BlockSpec((BM, n), lambda i: (i, 0)),
        out_shape=jax.ShapeDtypeStruct((m, n), F32),
    )(x, w, b.reshape(1, n))


def _mm_chunk(h, wl):
    m, k = h.shape
    n = wl.shape[1]
    nch = n // CW

    def body(h_ref, w_ref, o_ref):
        z = jnp.dot(h_ref[...], w_ref[...], preferred_element_type=F32)
        for c in range(nch):
            o_ref[c] = z[:, c * CW:(c + 1) * CW]

    return pl.pallas_call(
        body,
        grid=(m // BM,),
        in_specs=[
            pl.BlockSpec((BM, k), lambda i: (i, 0)),
            pl.BlockSpec((k, n), lambda i: (0, 0)),
        ],
        out_specs=pl.BlockSpec((nch, BM, CW), lambda i: (0, i, 0)),
        out_shape=jax.ShapeDtypeStruct((nch, m, CW), F32),
    )(h, wl)


def _combine(part, h_dst, wr, bl, inv, act):
    m, k = h_dst.shape
    nch = part.shape[0]
    n = nch * CW

    def body(p_ref, h_ref, w_ref, b_ref, i_ref, o_ref):
        s = jnp.concatenate(
            [p_ref[c, 0] + p_ref[c, 1] for c in range(nch)], axis=1)
        r = s * i_ref[:, :1] + jnp.dot(
            h_ref[...], w_ref[...], preferred_element_type=F32) + b_ref[...]
        o_ref[...] = jnp.maximum(r, 0.0) if act else r

    return pl.pallas_call(
        body,
        grid=(m // BM,),
        in_specs=[
            pl.BlockSpec((nch, NC, BM, CW), lambda i: (0, 0, i, 0)),
            pl.BlockSpec((BM, k), lambda i: (i, 0)),
            pl.BlockSpec((k, n), lambda i: (0, 0)),
            pl.BlockSpec((1, n), lambda i: (0, 0)),
            pl.BlockSpec((BM, 16), lambda i: (i, 0)),
        ],
        out_specs=pl.BlockSpec((BM, n), lambda i: (i, 0)),
        out_shape=jax.ShapeDtypeStruct((m, n), F32),
    )(part, h_dst, wr, bl.reshape(1, n), inv)


def _pair_dot(a, b):
    p, d = a.shape
    bp = p // 32

    def body(a_ref, b_ref, o_ref):
        o_ref[...] = jnp.sum(a_ref[...] * b_ref[...], axis=1, keepdims=True)

    return pl.pallas_call(
        body,
        grid=(32,),
        in_specs=[
            pl.BlockSpec((bp, d), lambda i: (i, 0)),
            pl.BlockSpec((bp, d), lambda i: (i, 0)),
        ],
        out_specs=pl.BlockSpec((bp, 1), lambda i: (i, 0)),
        out_shape=jax.ShapeDtypeStruct((p, 1), F32),
    )(a, b)


def _sc_counts(didx):
    e_pad = didx.shape[0]
    per_tile = e_pad // NS
    steps = per_tile // K_E
    mesh = plsc.VectorSubcoreMesh(core_axis_name="c", subcore_axis_name="s")

    @functools.partial(
        pl.kernel,
        out_type=jax.ShapeDtypeStruct((R_ACC, 16), F32),
        mesh=mesh,
        compiler_params=pltpu.CompilerParams(use_tc_tiling_on_sc=False),
        scratch_types=[
            pltpu.VMEM((K_E,), jnp.int32),
            pltpu.VMEM((K_E, 16), F32),
            pltpu.VMEM((RPT, 16), F32),
            pltpu.VMEM_SHARED((R_ACC, 16), F32),
            pltpu.SemaphoreType.DMA,
        ],
    )
    def k(didx_hbm, out_hbm, idx_v, ones_v, buf_v, acc_sh, sem):
        cid = lax.axis_index("c")
        sid = lax.axis_index("s")

        @pl.when(cid == 0)
        def _():
            def zrow(i, c):
                buf_v[i] = jnp.zeros((16,), F32)
                return c
            lax.fori_loop(0, RPT, zrow, 0)
            pltpu.sync_copy(buf_v, acc_sh.at[pl.ds(sid * RPT, RPT)])

            def orow(i, c):
                ones_v[i] = jnp.ones((16,), F32)
                return c
            lax.fori_loop(0, K_E, orow, 0)
            plsc.subcore_barrier()

            base = sid * per_tile

            def estep(j, c):
                pltpu.sync_copy(didx_hbm.at[pl.ds(base + j * K_E, K_E)], idx_v)
                pltpu.sync_copy(ones_v, acc_sh.at[idx_v], add=True)
                return c
            lax.fori_loop(0, steps, estep, 0)
            plsc.subcore_barrier()

            pltpu.sync_copy(acc_sh.at[pl.ds(sid * RPT, RPT)], buf_v)

            def irow(i, c):
                buf_v[i] = 1.0 / jnp.maximum(buf_v[i], 1.0)
                return c
            lax.fori_loop(0, RPT, irow, 0)
            pltpu.sync_copy(buf_v, out_hbm.at[pl.ds(sid * RPT, RPT)])

    return k(didx)


def _sc_agg(z, sidx, didx):
    nch = z.shape[0]
    e_pad = sidx.shape[0]
    per_tile = e_pad // NW
    steps = per_tile // K_E
    mesh = plsc.VectorSubcoreMesh(core_axis_name="c", subcore_axis_name="s")

    @functools.partial(
        pl.kernel,
        out_type=jax.ShapeDtypeStruct((nch, NC, R_ACC, CW), F32),
        mesh=mesh,
        compiler_params=pltpu.CompilerParams(use_tc_tiling_on_sc=False),
        scratch_types=[
            pltpu.VMEM((K_E,), jnp.int32),
            pltpu.VMEM((K_E,), jnp.int32),
            pltpu.VMEM((K_E, CW), F32),
            pltpu.VMEM((ZB, CW), F32),
            pltpu.VMEM_SHARED((R_ACC, CW), F32),
            pltpu.SemaphoreType.DMA,
        ],
    )
    def k(z_hbm, sidx_hbm, didx_hbm, out_hbm, sidx_v, didx_v, rows_v,
          zbuf_v, acc_sh, sem):
        cid = lax.axis_index("c")
        sid = lax.axis_index("s")
        wid = cid * NS + sid
        base = wid * per_tile

        def zrow(i, c):
            zbuf_v[i, 0:16] = jnp.zeros((16,), F32)
            zbuf_v[i, 16:32] = jnp.zeros((16,), F32)
            return c
        lax.fori_loop(0, ZB, zrow, 0)

        for ch in range(nch):
            def zacc(t, c):
                pltpu.sync_copy(
                    zbuf_v, acc_sh.at[pl.ds(sid * RPT + t * ZB, ZB)])
                return c
            lax.fori_loop(0, RPT // ZB, zacc, 0)
            plsc.subcore_barrier()

            def estep(j, c):
                off = base + j * K_E
                pltpu.sync_copy(sidx_hbm.at[pl.ds(off, K_E)], sidx_v)
                pltpu.sync_copy(didx_hbm.at[pl.ds(off, K_E)], didx_v)
                pltpu.async_copy(z_hbm.at[ch].at[sidx_v], rows_v, sem).wait()
                pltpu.sync_copy(rows_v, acc_sh.at[didx_v], add=True)
                return c
            lax.fori_loop(0, steps, estep, 0)
            plsc.subcore_barrier()

            pltpu.sync_copy(
                acc_sh.at[pl.ds(sid * RPT, RPT)],
                out_hbm.at[ch].at[cid].at[pl.ds(sid * RPT, RPT)])
            plsc.subcore_barrier()

    return k(z, sidx, didx)


def _sc_pair_gather(h_a, h_b, idx_a, idx_b):
    p = idx_a.shape[0]
    d = h_a.shape[1]
    per_tile = p // NW
    steps = per_tile // K_E
    mesh = plsc.VectorSubcoreMesh(core_axis_name="c", subcore_axis_name="s")

    @functools.partial(
        pl.kernel,
        out_type=[jax.ShapeDtypeStruct((p, d), F32),
                  jax.ShapeDtypeStruct((p, d), F32)],
        mesh=mesh,
        compiler_params=pltpu.CompilerParams(use_tc_tiling_on_sc=False),
        scratch_types=[
            pltpu.VMEM((K_E,), jnp.int32),
            pltpu.VMEM((K_E, d), F32),
            pltpu.SemaphoreType.DMA,
        ],
    )
    def k(ha_hbm, hb_hbm, ia_hbm, ib_hbm, oa_hbm, ob_hbm, idx_v, rows_v, sem):
        cid = lax.axis_index("c")
        sid = lax.axis_index("s")
        base = (cid * NS + sid) * per_tile

        def step(j, c):
            off = base + j * K_E
            pltpu.sync_copy(ia_hbm.at[pl.ds(off, K_E)], idx_v)
            pltpu.async_copy(ha_hbm.at[idx_v], rows_v, sem).wait()
            pltpu.sync_copy(rows_v, oa_hbm.at[pl.ds(off, K_E)])
            pltpu.sync_copy(ib_hbm.at[pl.ds(off, K_E)], idx_v)
            pltpu.async_copy(hb_hbm.at[idx_v], rows_v, sem).wait()
            pltpu.sync_copy(rows_v, ob_hbm.at[pl.ds(off, K_E)])
            return c
        lax.fori_loop(0, steps, step, 0)

    return k(h_a, h_b, idx_a, idx_b)


def _pad_edges(ei, pad_dst):
    e = ei.shape[1]
    e_pad = ((e + NW * K_E - 1) // (NW * K_E)) * (NW * K_E)
    sidx = jnp.concatenate(
        [ei[0], jnp.zeros((e_pad - e,), jnp.int32)])
    didx = jnp.concatenate(
        [ei[1], jnp.full((e_pad - e,), pad_dst, jnp.int32)])
    return sidx, didx


def kernel(x_drug, x_disorder, edge_index_drug_to_disorder,
           edge_index_disorder_to_drug, edge_label_index, params):
    pad_row = 50000
    s_d2s, d_d2s = _pad_edges(edge_index_drug_to_disorder, pad_row)
    s_s2d, d_s2d = _pad_edges(edge_index_disorder_to_drug, pad_row)

    h_dr = _mm_bias(x_drug, params["W_drug"], params["b_drug"])
    h_di = _mm_bias(x_disorder, params["W_disorder"], params["b_disorder"])

    inv_di = _sc_counts(d_d2s)
    inv_dr = _sc_counts(d_s2d)

    n_layers = len(params["layers"])
    for i, lp in enumerate(params["layers"]):
        act = i < n_layers - 1
        z_d2s = _mm_chunk(h_dr, lp["Wl_d2s"])
        z_s2d = _mm_chunk(h_di, lp["Wl_s2d"])
        p_d2s = _sc_agg(z_d2s, s_d2s, d_d2s)
        p_s2d = _sc_agg(z_s2d, s_s2d, d_s2d)
        new_di = _combine(p_d2s, h_di, lp["Wr_d2s"], lp["bl_d2s"], inv_di, act)
        new_dr = _combine(p_s2d, h_dr, lp["Wr_s2d"], lp["bl_s2d"], inv_dr, act)
        h_dr, h_di = new_dr, new_di

    ef_a, ef_b = _sc_pair_gather(
        h_dr, h_di, edge_label_index[0], edge_label_index[1])
    return _pair_dot(ef_a, ef_b).reshape(-1)
